# baseline (speedup 1.0000x reference)
import functools

import jax
import jax.numpy as jnp
from jax import lax
from jax.experimental import pallas as pl
from jax.experimental.pallas import tpu as pltpu
from jax.experimental.pallas import tpu_sc as plsc

V = 1000000
D = 64
B = 16384
K = 10

NC = 2
NS = 16
NW = NC * NS
BPW = B // NW
SCR = 64
SUP = BPW // SCR
L = 16
NCHK = D // L

PBLK = 256
H = 500224
NPB = H // PBLK


def _pack_body(a_ref, b_ref, out_ref):
    eye = jnp.eye(D, dtype=jnp.float32)
    at = lax.dot_general(a_ref[...], eye, (((0,), (0,)), ((), ())),
                         preferred_element_type=jnp.float32)
    bt = lax.dot_general(b_ref[...], eye, (((0,), (0,)), ((), ())),
                         preferred_element_type=jnp.float32)
    out_ref[...] = jnp.concatenate([at, bt], axis=1)


def _tc_pack(table):
    tt = table.T
    return pl.pallas_call(
        _pack_body,
        grid=(NPB,),
        in_specs=[
            pl.BlockSpec((D, PBLK), lambda i: (0, i)),
            pl.BlockSpec((D, PBLK),
                         lambda i: (0, jnp.minimum(NPB + i, (V - 1) // PBLK))),
        ],
        out_specs=pl.BlockSpec((PBLK, 2 * D), lambda i: (i, 0)),
        out_shape=jax.ShapeDtypeStruct((H, 2 * D), jnp.float32),
    )(tt, tt)


def _sc_body(center_hbm, pos_hbm, negf_hbm, in_pk, out_pk,
             pacc_out, nacc_out,
             row_c, off_c, row_p, off_p, row_n, off_n,
             v_rows, up_rows, un_rows, pacc_buf, nacc_buf, sem):
    wid = lax.axis_index("s") * NC + lax.axis_index("c")
    base = wid * BPW

    pltpu.sync_copy(center_hbm.at[pl.ds(base, BPW)], row_c)
    pltpu.sync_copy(pos_hbm.at[pl.ds(base, BPW)], row_p)
    pltpu.sync_copy(negf_hbm.at[pl.ds(base * K, BPW * K)], row_n)

    def split(row_ref, off_ref, n):
        def step(i, carry):
            idx = row_ref[pl.ds(i * L, L)]
            hi = idx >= H
            row_ref[pl.ds(i * L, L)] = jnp.where(hi, idx - H, idx)
            off_ref[pl.ds(i * L, L)] = jnp.where(hi, D, 0)
            return carry
        lax.fori_loop(0, n // L, step, 0)

    split(row_c, off_c, BPW)
    split(row_p, off_p, BPW)
    split(row_n, off_n, BPW * K)

    def super_chunk(c, carry):
        cps = [
            pltpu.async_copy(in_pk.at[row_c.at[pl.ds(c * SCR, SCR)]], v_rows, sem),
            pltpu.async_copy(out_pk.at[row_p.at[pl.ds(c * SCR, SCR)]], up_rows, sem),
        ]
        for q in range(K):
            cps.append(pltpu.async_copy(
                out_pk.at[row_n.at[pl.ds(c * SCR * K + q * SCR, SCR)]],
                un_rows.at[pl.ds(q * SCR, SCR), :], sem))
        for cp in cps:
            cp.wait()

        def group16(g, carry2):
            gbase = c * SCR + g * L
            ocv = off_c[pl.ds(gbase, L)]
            opv = off_p[pl.ds(gbase, L)]
            onvs = [off_n[pl.ds(gbase * K + j * L, L)] for j in range(K)]
            for r16 in range(L):
                r = g * L + r16
                oc = ocv[r16]
                op = opv[r16]
                vc = [v_rows[r, pl.ds(oc + ch * L, L)] for ch in range(NCHK)]
                uc = [up_rows[r, pl.ds(op + ch * L, L)] for ch in range(NCHK)]
                p = vc[0] * uc[0]
                for ch in range(1, NCHK):
                    p = p + vc[ch] * uc[ch]
                pacc_buf[pl.ds(r * L, L)] = p
                rK = r * K
                for q in range(K):
                    t = r16 * K + q
                    on = onvs[t // L][t % L]
                    nc_ = [un_rows[rK + q, pl.ds(on + ch * L, L)] for ch in range(NCHK)]
                    n = vc[0] * nc_[0]
                    for ch in range(1, NCHK):
                        n = n + vc[ch] * nc_[ch]
                    nacc_buf[pl.ds((rK + q) * L, L)] = -n
            return carry2

        lax.fori_loop(0, SCR // L, group16, carry)
        pltpu.sync_copy(pacc_buf,
                        pacc_out.at[pl.ds((base + c * SCR) * L, SCR * L)])
        pltpu.sync_copy(nacc_buf,
                        nacc_out.at[pl.ds((base * K + c * SCR * K) * L, SCR * K * L)])
        return carry

    lax.fori_loop(0, SUP, super_chunk, 0)


def _make_sc_scores():
    mesh = plsc.VectorSubcoreMesh(core_axis_name="c", subcore_axis_name="s")
    return functools.partial(
        pl.kernel,
        out_type=(
            jax.ShapeDtypeStruct((B * L,), jnp.float32),
            jax.ShapeDtypeStruct((B * K * L,), jnp.float32),
        ),
        mesh=mesh,
        scratch_types=[
            pltpu.VMEM((BPW,), jnp.int32),
            pltpu.VMEM((BPW,), jnp.int32),
            pltpu.VMEM((BPW,), jnp.int32),
            pltpu.VMEM((BPW,), jnp.int32),
            pltpu.VMEM((BPW * K,), jnp.int32),
            pltpu.VMEM((BPW * K,), jnp.int32),
            pltpu.VMEM((SCR, 2 * D), jnp.float32),
            pltpu.VMEM((SCR, 2 * D), jnp.float32),
            pltpu.VMEM((SCR * K, 2 * D), jnp.float32),
            pltpu.VMEM((SCR * L,), jnp.float32),
            pltpu.VMEM((SCR * K * L,), jnp.float32),
            pltpu.SemaphoreType.DMA,
        ],
        compiler_params=pltpu.CompilerParams(use_tc_tiling_on_sc=False,
                                             needs_layout_passes=False),
    )(_sc_body)


def _tc_loss_body(pa_ref, na_ref, out_ref):
    i = lax.broadcasted_iota(jnp.int32, (128, 128), 0)
    j = lax.broadcasted_iota(jnp.int32, (128, 128), 1)
    S = jnp.where(i // L == j, 1.0, 0.0).astype(jnp.float32)
    lane = lax.broadcasted_iota(jnp.int32, (1, 128), 1)
    mask = lane < (128 // L)

    ps = jnp.dot(pa_ref[...], S, preferred_element_type=jnp.float32)
    lp = jnp.where(mask, jnp.log(1.0 / (1.0 + jnp.exp(-ps)) + 1e-9), 0.0)
    ns = jnp.dot(na_ref[...], S, preferred_element_type=jnp.float32)
    ln = jnp.where(mask, jnp.log(1.0 / (1.0 + jnp.exp(-ns)) + 1e-9), 0.0)
    out_ref[0, 0] = -(jnp.sum(lp) / B + jnp.sum(ln) / (B * K))


def _tc_loss(pacc, nacc):
    return pl.pallas_call(
        _tc_loss_body,
        out_specs=pl.BlockSpec(memory_space=pltpu.SMEM),
        out_shape=jax.ShapeDtypeStruct((1, 1), jnp.float32),
    )(pacc.reshape(B * L // 128, 128), nacc.reshape(B * K * L // 128, 128))


def kernel(center, pos, neg, in_embed, out_embed):
    center = center.astype(jnp.int32)
    pos = pos.astype(jnp.int32)
    neg_flat = neg.astype(jnp.int32).reshape(B * K)
    in_pk = _tc_pack(in_embed)
    out_pk = _tc_pack(out_embed)
    pacc, nacc = _make_sc_scores()(center, pos, neg_flat, in_pk, out_pk)
    return _tc_loss(pacc, nacc)[0, 0]


# speedup vs baseline: 3.2399x; 3.2399x over previous
import functools

import jax
import jax.numpy as jnp
from jax import lax
from jax.experimental import pallas as pl
from jax.experimental.pallas import tpu as pltpu
from jax.experimental.pallas import tpu_sc as plsc

V = 1000000
D = 64
B = 16384
K = 10

NC = 2
NS = 16
NW = NC * NS
BPW = B // NW
SCR = 64
SUP = BPW // SCR
L = 16
NCHK = D // L

PBLK = 2048
H = 501760
NPB = H // PBLK


def _pack_body(a_ref, b_ref, out_ref):
    eye = jnp.eye(D, dtype=jnp.float32)
    at = lax.dot_general(a_ref[...], eye, (((0,), (0,)), ((), ())),
                         preferred_element_type=jnp.float32)
    bt = lax.dot_general(b_ref[...], eye, (((0,), (0,)), ((), ())),
                         preferred_element_type=jnp.float32)
    out_ref[...] = jnp.concatenate([at, bt], axis=1)


def _tc_pack(table):
    tt = table.T
    return pl.pallas_call(
        _pack_body,
        grid=(NPB,),
        in_specs=[
            pl.BlockSpec((D, PBLK), lambda i: (0, i)),
            pl.BlockSpec((D, PBLK),
                         lambda i: (0, jnp.minimum(NPB + i, (V - 1) // PBLK))),
        ],
        out_specs=pl.BlockSpec((PBLK, 2 * D), lambda i: (i, 0)),
        out_shape=jax.ShapeDtypeStruct((H, 2 * D), jnp.float32),
    )(tt, tt)


def _sc_body(center_hbm, pos_hbm, negf_hbm, in_pk, out_pk,
             pacc_out, nacc_out,
             row_c, off_c, row_p, off_p, row_n, off_n,
             v_rows, up_rows, un_rows, pacc_buf, nacc_buf, sem):
    wid = lax.axis_index("s") * NC + lax.axis_index("c")
    base = wid * BPW

    pltpu.sync_copy(center_hbm.at[pl.ds(base, BPW)], row_c)
    pltpu.sync_copy(pos_hbm.at[pl.ds(base, BPW)], row_p)
    pltpu.sync_copy(negf_hbm.at[pl.ds(base * K, BPW * K)], row_n)

    def split(row_ref, off_ref, n):
        def step(i, carry):
            idx = row_ref[pl.ds(i * L, L)]
            hi = idx >= H
            row_ref[pl.ds(i * L, L)] = jnp.where(hi, idx - H, idx)
            off_ref[pl.ds(i * L, L)] = jnp.where(hi, D, 0)
            return carry
        lax.fori_loop(0, n // L, step, 0)

    split(row_c, off_c, BPW)
    split(row_p, off_p, BPW)
    split(row_n, off_n, BPW * K)

    def super_chunk(c, carry):
        cps = [
            pltpu.async_copy(in_pk.at[row_c.at[pl.ds(c * SCR, SCR)]], v_rows, sem),
            pltpu.async_copy(out_pk.at[row_p.at[pl.ds(c * SCR, SCR)]], up_rows, sem),
        ]
        for q in range(K):
            cps.append(pltpu.async_copy(
                out_pk.at[row_n.at[pl.ds(c * SCR * K + q * SCR, SCR)]],
                un_rows.at[pl.ds(q * SCR, SCR), :], sem))
        for cp in cps:
            cp.wait()

        def group16(g, carry2):
            gbase = c * SCR + g * L
            ocv = off_c[pl.ds(gbase, L)]
            opv = off_p[pl.ds(gbase, L)]
            onvs = [off_n[pl.ds(gbase * K + j * L, L)] for j in range(K)]
            for r16 in range(L):
                r = g * L + r16
                oc = ocv[r16]
                op = opv[r16]
                vc = [v_rows[r, pl.ds(oc + ch * L, L)] for ch in range(NCHK)]
                uc = [up_rows[r, pl.ds(op + ch * L, L)] for ch in range(NCHK)]
                p = vc[0] * uc[0]
                for ch in range(1, NCHK):
                    p = p + vc[ch] * uc[ch]
                pacc_buf[pl.ds(r * L, L)] = p
                rK = r * K
                for q in range(K):
                    t = r16 * K + q
                    on = onvs[t // L][t % L]
                    nc_ = [un_rows[rK + q, pl.ds(on + ch * L, L)] for ch in range(NCHK)]
                    n = vc[0] * nc_[0]
                    for ch in range(1, NCHK):
                        n = n + vc[ch] * nc_[ch]
                    nacc_buf[pl.ds((rK + q) * L, L)] = -n
            return carry2

        lax.fori_loop(0, SCR // L, group16, carry)
        pltpu.sync_copy(pacc_buf,
                        pacc_out.at[pl.ds((base + c * SCR) * L, SCR * L)])
        pltpu.sync_copy(nacc_buf,
                        nacc_out.at[pl.ds((base * K + c * SCR * K) * L, SCR * K * L)])
        return carry

    lax.fori_loop(0, SUP, super_chunk, 0)


def _make_sc_scores():
    mesh = plsc.VectorSubcoreMesh(core_axis_name="c", subcore_axis_name="s")
    return functools.partial(
        pl.kernel,
        out_type=(
            jax.ShapeDtypeStruct((B * L,), jnp.float32),
            jax.ShapeDtypeStruct((B * K * L,), jnp.float32),
        ),
        mesh=mesh,
        scratch_types=[
            pltpu.VMEM((BPW,), jnp.int32),
            pltpu.VMEM((BPW,), jnp.int32),
            pltpu.VMEM((BPW,), jnp.int32),
            pltpu.VMEM((BPW,), jnp.int32),
            pltpu.VMEM((BPW * K,), jnp.int32),
            pltpu.VMEM((BPW * K,), jnp.int32),
            pltpu.VMEM((SCR, 2 * D), jnp.float32),
            pltpu.VMEM((SCR, 2 * D), jnp.float32),
            pltpu.VMEM((SCR * K, 2 * D), jnp.float32),
            pltpu.VMEM((SCR * L,), jnp.float32),
            pltpu.VMEM((SCR * K * L,), jnp.float32),
            pltpu.SemaphoreType.DMA,
        ],
        compiler_params=pltpu.CompilerParams(use_tc_tiling_on_sc=False,
                                             needs_layout_passes=False),
    )(_sc_body)


def _tc_loss_body(pa_ref, na_ref, out_ref):
    i = lax.broadcasted_iota(jnp.int32, (128, 128), 0)
    j = lax.broadcasted_iota(jnp.int32, (128, 128), 1)
    S = jnp.where(i // L == j, 1.0, 0.0).astype(jnp.float32)
    lane = lax.broadcasted_iota(jnp.int32, (1, 128), 1)
    mask = lane < (128 // L)

    ps = jnp.dot(pa_ref[...], S, preferred_element_type=jnp.float32)
    lp = jnp.where(mask, jnp.log(1.0 / (1.0 + jnp.exp(-ps)) + 1e-9), 0.0)
    ns = jnp.dot(na_ref[...], S, preferred_element_type=jnp.float32)
    ln = jnp.where(mask, jnp.log(1.0 / (1.0 + jnp.exp(-ns)) + 1e-9), 0.0)
    out_ref[0, 0] = -(jnp.sum(lp) / B + jnp.sum(ln) / (B * K))


def _tc_loss(pacc, nacc):
    return pl.pallas_call(
        _tc_loss_body,
        out_specs=pl.BlockSpec(memory_space=pltpu.SMEM),
        out_shape=jax.ShapeDtypeStruct((1, 1), jnp.float32),
    )(pacc.reshape(B * L // 128, 128), nacc.reshape(B * K * L // 128, 128))


def kernel(center, pos, neg, in_embed, out_embed):
    center = center.astype(jnp.int32)
    pos = pos.astype(jnp.int32)
    neg_flat = neg.astype(jnp.int32).reshape(B * K)
    in_pk = _tc_pack(in_embed)
    out_pk = _tc_pack(out_embed)
    pacc, nacc = _make_sc_scores()(center, pos, neg_flat, in_pk, out_pk)
    return _tc_loss(pacc, nacc)[0, 0]


# speedup vs baseline: 4.2492x; 1.3115x over previous
import functools

import jax
import jax.numpy as jnp
from jax import lax
from jax.experimental import pallas as pl
from jax.experimental.pallas import tpu as pltpu
from jax.experimental.pallas import tpu_sc as plsc

V = 1000000
D = 64
B = 16384
K = 10

NC = 2
NS = 16
NW = NC * NS
BPW = B // NW
SCR = 64
SUP = BPW // SCR
L = 16
NCHK = D // L

PBLK = 8192
H = 507904
NPB = H // PBLK


def _pack_body(a_ref, b_ref, out_ref):
    eye = jnp.eye(D, dtype=jnp.float32)
    at = lax.dot_general(a_ref[...], eye, (((0,), (0,)), ((), ())),
                         preferred_element_type=jnp.float32)
    bt = lax.dot_general(b_ref[...], eye, (((0,), (0,)), ((), ())),
                         preferred_element_type=jnp.float32)
    out_ref[...] = jnp.concatenate([at, bt], axis=1)


def _tc_pack(table):
    tt = table.T
    return pl.pallas_call(
        _pack_body,
        grid=(NPB,),
        in_specs=[
            pl.BlockSpec((D, PBLK), lambda i: (0, i)),
            pl.BlockSpec((D, PBLK),
                         lambda i: (0, jnp.minimum(NPB + i, (V - 1) // PBLK))),
        ],
        out_specs=pl.BlockSpec((PBLK, 2 * D), lambda i: (i, 0)),
        out_shape=jax.ShapeDtypeStruct((H, 2 * D), jnp.float32),
    )(tt, tt)


def _sc_body(center_hbm, pos_hbm, negf_hbm, in_pk, out_pk,
             pacc_out, nacc_out,
             row_c, off_c, row_p, off_p, row_n, off_n,
             v_rows, up_rows, un_rows, pacc_buf, nacc_buf, sem):
    wid = lax.axis_index("s") * NC + lax.axis_index("c")
    base = wid * BPW

    pltpu.sync_copy(center_hbm.at[pl.ds(base, BPW)], row_c)
    pltpu.sync_copy(pos_hbm.at[pl.ds(base, BPW)], row_p)
    pltpu.sync_copy(negf_hbm.at[pl.ds(base * K, BPW * K)], row_n)

    def split(row_ref, off_ref, n):
        def step(i, carry):
            idx = row_ref[pl.ds(i * L, L)]
            hi = idx >= H
            row_ref[pl.ds(i * L, L)] = jnp.where(hi, idx - H, idx)
            off_ref[pl.ds(i * L, L)] = jnp.where(hi, D, 0)
            return carry
        lax.fori_loop(0, n // L, step, 0)

    split(row_c, off_c, BPW)
    split(row_p, off_p, BPW)
    split(row_n, off_n, BPW * K)

    def super_chunk(c, carry):
        cps = [
            pltpu.async_copy(in_pk.at[row_c.at[pl.ds(c * SCR, SCR)]], v_rows, sem),
            pltpu.async_copy(out_pk.at[row_p.at[pl.ds(c * SCR, SCR)]], up_rows, sem),
        ]
        for q in range(K):
            cps.append(pltpu.async_copy(
                out_pk.at[row_n.at[pl.ds(c * SCR * K + q * SCR, SCR)]],
                un_rows.at[pl.ds(q * SCR, SCR), :], sem))
        for cp in cps:
            cp.wait()

        def group16(g, carry2):
            gbase = c * SCR + g * L
            ocv = off_c[pl.ds(gbase, L)]
            opv = off_p[pl.ds(gbase, L)]
            onvs = [off_n[pl.ds(gbase * K + j * L, L)] for j in range(K)]
            for r16 in range(L):
                r = g * L + r16
                oc = ocv[r16]
                op = opv[r16]
                vc = [v_rows[r, pl.ds(oc + ch * L, L)] for ch in range(NCHK)]
                uc = [up_rows[r, pl.ds(op + ch * L, L)] for ch in range(NCHK)]
                p = vc[0] * uc[0]
                for ch in range(1, NCHK):
                    p = p + vc[ch] * uc[ch]
                pacc_buf[pl.ds(r * L, L)] = p
                rK = r * K
                for q in range(K):
                    t = r16 * K + q
                    on = onvs[t // L][t % L]
                    nc_ = [un_rows[rK + q, pl.ds(on + ch * L, L)] for ch in range(NCHK)]
                    n = vc[0] * nc_[0]
                    for ch in range(1, NCHK):
                        n = n + vc[ch] * nc_[ch]
                    nacc_buf[pl.ds((rK + q) * L, L)] = -n
            return carry2

        lax.fori_loop(0, SCR // L, group16, carry)
        pltpu.sync_copy(pacc_buf,
                        pacc_out.at[pl.ds((base + c * SCR) * L, SCR * L)])
        pltpu.sync_copy(nacc_buf,
                        nacc_out.at[pl.ds((base * K + c * SCR * K) * L, SCR * K * L)])
        return carry

    lax.fori_loop(0, SUP, super_chunk, 0)


def _make_sc_scores():
    mesh = plsc.VectorSubcoreMesh(core_axis_name="c", subcore_axis_name="s")
    return functools.partial(
        pl.kernel,
        out_type=(
            jax.ShapeDtypeStruct((B * L,), jnp.float32),
            jax.ShapeDtypeStruct((B * K * L,), jnp.float32),
        ),
        mesh=mesh,
        scratch_types=[
            pltpu.VMEM((BPW,), jnp.int32),
            pltpu.VMEM((BPW,), jnp.int32),
            pltpu.VMEM((BPW,), jnp.int32),
            pltpu.VMEM((BPW,), jnp.int32),
            pltpu.VMEM((BPW * K,), jnp.int32),
            pltpu.VMEM((BPW * K,), jnp.int32),
            pltpu.VMEM((SCR, 2 * D), jnp.float32),
            pltpu.VMEM((SCR, 2 * D), jnp.float32),
            pltpu.VMEM((SCR * K, 2 * D), jnp.float32),
            pltpu.VMEM((SCR * L,), jnp.float32),
            pltpu.VMEM((SCR * K * L,), jnp.float32),
            pltpu.SemaphoreType.DMA,
        ],
        compiler_params=pltpu.CompilerParams(use_tc_tiling_on_sc=False,
                                             needs_layout_passes=False),
    )(_sc_body)


def _tc_loss_body(pa_ref, na_ref, out_ref):
    i = lax.broadcasted_iota(jnp.int32, (128, 128), 0)
    j = lax.broadcasted_iota(jnp.int32, (128, 128), 1)
    S = jnp.where(i // L == j, 1.0, 0.0).astype(jnp.float32)
    lane = lax.broadcasted_iota(jnp.int32, (1, 128), 1)
    mask = lane < (128 // L)

    ps = jnp.dot(pa_ref[...], S, preferred_element_type=jnp.float32)
    lp = jnp.where(mask, jnp.log(1.0 / (1.0 + jnp.exp(-ps)) + 1e-9), 0.0)
    ns = jnp.dot(na_ref[...], S, preferred_element_type=jnp.float32)
    ln = jnp.where(mask, jnp.log(1.0 / (1.0 + jnp.exp(-ns)) + 1e-9), 0.0)
    out_ref[0, 0] = -(jnp.sum(lp) / B + jnp.sum(ln) / (B * K))


def _tc_loss(pacc, nacc):
    return pl.pallas_call(
        _tc_loss_body,
        out_specs=pl.BlockSpec(memory_space=pltpu.SMEM),
        out_shape=jax.ShapeDtypeStruct((1, 1), jnp.float32),
    )(pacc.reshape(B * L // 128, 128), nacc.reshape(B * K * L // 128, 128))


def kernel(center, pos, neg, in_embed, out_embed):
    center = center.astype(jnp.int32)
    pos = pos.astype(jnp.int32)
    neg_flat = neg.astype(jnp.int32).reshape(B * K)
    in_pk = _tc_pack(in_embed)
    out_pk = _tc_pack(out_embed)
    pacc, nacc = _make_sc_scores()(center, pos, neg_flat, in_pk, out_pk)
    return _tc_loss(pacc, nacc)[0, 0]


# speedup vs baseline: 4.4516x; 1.0476x over previous
import functools

import jax
import jax.numpy as jnp
from jax import lax
from jax.experimental import pallas as pl
from jax.experimental.pallas import tpu as pltpu
from jax.experimental.pallas import tpu_sc as plsc

V = 1000000
D = 64
B = 16384
K = 10

NC = 2
NS = 16
NW = NC * NS
BPW = B // NW
SCR = 64
SUP = BPW // SCR
L = 16
NCHK = D // L

PBLK = 16384
H = 507904
NPB = H // PBLK


def _pack_body(a_ref, b_ref, out_ref):
    eye = jnp.eye(D, dtype=jnp.float32)
    at = lax.dot_general(a_ref[...], eye, (((0,), (0,)), ((), ())),
                         preferred_element_type=jnp.float32)
    bt = lax.dot_general(b_ref[...], eye, (((0,), (0,)), ((), ())),
                         preferred_element_type=jnp.float32)
    out_ref[...] = jnp.concatenate([at, bt], axis=1)


def _tc_pack(table):
    tt = table.T
    return pl.pallas_call(
        _pack_body,
        grid=(NPB,),
        in_specs=[
            pl.BlockSpec((D, PBLK), lambda i: (0, i)),
            pl.BlockSpec((D, PBLK),
                         lambda i: (0, jnp.minimum(NPB + i, (V - 1) // PBLK))),
        ],
        out_specs=pl.BlockSpec((PBLK, 2 * D), lambda i: (i, 0)),
        out_shape=jax.ShapeDtypeStruct((H, 2 * D), jnp.float32),
    )(tt, tt)


def _sc_body(center_hbm, pos_hbm, negf_hbm, in_pk, out_pk,
             pacc_out, nacc_out,
             row_c, off_c, row_p, off_p, row_n, off_n,
             v_rows, up_rows, un_rows, pacc_buf, nacc_buf, sem):
    wid = lax.axis_index("s") * NC + lax.axis_index("c")
    base = wid * BPW

    pltpu.sync_copy(center_hbm.at[pl.ds(base, BPW)], row_c)
    pltpu.sync_copy(pos_hbm.at[pl.ds(base, BPW)], row_p)
    pltpu.sync_copy(negf_hbm.at[pl.ds(base * K, BPW * K)], row_n)

    def split(row_ref, off_ref, n):
        def step(i, carry):
            idx = row_ref[pl.ds(i * L, L)]
            hi = idx >= H
            row_ref[pl.ds(i * L, L)] = jnp.where(hi, idx - H, idx)
            off_ref[pl.ds(i * L, L)] = jnp.where(hi, D, 0)
            return carry
        lax.fori_loop(0, n // L, step, 0)

    split(row_c, off_c, BPW)
    split(row_p, off_p, BPW)
    split(row_n, off_n, BPW * K)

    def super_chunk(c, carry):
        cps = [
            pltpu.async_copy(in_pk.at[row_c.at[pl.ds(c * SCR, SCR)]], v_rows, sem),
            pltpu.async_copy(out_pk.at[row_p.at[pl.ds(c * SCR, SCR)]], up_rows, sem),
        ]
        for q in range(K):
            cps.append(pltpu.async_copy(
                out_pk.at[row_n.at[pl.ds(c * SCR * K + q * SCR, SCR)]],
                un_rows.at[pl.ds(q * SCR, SCR), :], sem))
        for cp in cps:
            cp.wait()

        def group16(g, carry2):
            gbase = c * SCR + g * L
            ocv = off_c[pl.ds(gbase, L)]
            opv = off_p[pl.ds(gbase, L)]
            onvs = [off_n[pl.ds(gbase * K + j * L, L)] for j in range(K)]
            for r16 in range(L):
                r = g * L + r16
                oc = ocv[r16]
                op = opv[r16]
                vc = [v_rows[r, pl.ds(oc + ch * L, L)] for ch in range(NCHK)]
                uc = [up_rows[r, pl.ds(op + ch * L, L)] for ch in range(NCHK)]
                p = vc[0] * uc[0]
                for ch in range(1, NCHK):
                    p = p + vc[ch] * uc[ch]
                pacc_buf[pl.ds(r * L, L)] = p
                rK = r * K
                for q in range(K):
                    t = r16 * K + q
                    on = onvs[t // L][t % L]
                    nc_ = [un_rows[rK + q, pl.ds(on + ch * L, L)] for ch in range(NCHK)]
                    n = vc[0] * nc_[0]
                    for ch in range(1, NCHK):
                        n = n + vc[ch] * nc_[ch]
                    nacc_buf[pl.ds((rK + q) * L, L)] = -n
            return carry2

        lax.fori_loop(0, SCR // L, group16, carry)
        pltpu.sync_copy(pacc_buf,
                        pacc_out.at[pl.ds((base + c * SCR) * L, SCR * L)])
        pltpu.sync_copy(nacc_buf,
                        nacc_out.at[pl.ds((base * K + c * SCR * K) * L, SCR * K * L)])
        return carry

    lax.fori_loop(0, SUP, super_chunk, 0)


def _make_sc_scores():
    mesh = plsc.VectorSubcoreMesh(core_axis_name="c", subcore_axis_name="s")
    return functools.partial(
        pl.kernel,
        out_type=(
            jax.ShapeDtypeStruct((B * L,), jnp.float32),
            jax.ShapeDtypeStruct((B * K * L,), jnp.float32),
        ),
        mesh=mesh,
        scratch_types=[
            pltpu.VMEM((BPW,), jnp.int32),
            pltpu.VMEM((BPW,), jnp.int32),
            pltpu.VMEM((BPW,), jnp.int32),
            pltpu.VMEM((BPW,), jnp.int32),
            pltpu.VMEM((BPW * K,), jnp.int32),
            pltpu.VMEM((BPW * K,), jnp.int32),
            pltpu.VMEM((SCR, 2 * D), jnp.float32),
            pltpu.VMEM((SCR, 2 * D), jnp.float32),
            pltpu.VMEM((SCR * K, 2 * D), jnp.float32),
            pltpu.VMEM((SCR * L,), jnp.float32),
            pltpu.VMEM((SCR * K * L,), jnp.float32),
            pltpu.SemaphoreType.DMA,
        ],
        compiler_params=pltpu.CompilerParams(use_tc_tiling_on_sc=False,
                                             needs_layout_passes=False),
    )(_sc_body)


def _tc_loss_body(pa_ref, na_ref, out_ref):
    i = lax.broadcasted_iota(jnp.int32, (128, 128), 0)
    j = lax.broadcasted_iota(jnp.int32, (128, 128), 1)
    S = jnp.where(i // L == j, 1.0, 0.0).astype(jnp.float32)
    lane = lax.broadcasted_iota(jnp.int32, (1, 128), 1)
    mask = lane < (128 // L)

    ps = jnp.dot(pa_ref[...], S, preferred_element_type=jnp.float32)
    lp = jnp.where(mask, jnp.log(1.0 / (1.0 + jnp.exp(-ps)) + 1e-9), 0.0)
    ns = jnp.dot(na_ref[...], S, preferred_element_type=jnp.float32)
    ln = jnp.where(mask, jnp.log(1.0 / (1.0 + jnp.exp(-ns)) + 1e-9), 0.0)
    out_ref[0, 0] = -(jnp.sum(lp) / B + jnp.sum(ln) / (B * K))


def _tc_loss(pacc, nacc):
    return pl.pallas_call(
        _tc_loss_body,
        out_specs=pl.BlockSpec(memory_space=pltpu.SMEM),
        out_shape=jax.ShapeDtypeStruct((1, 1), jnp.float32),
    )(pacc.reshape(B * L // 128, 128), nacc.reshape(B * K * L // 128, 128))


def kernel(center, pos, neg, in_embed, out_embed):
    center = center.astype(jnp.int32)
    pos = pos.astype(jnp.int32)
    neg_flat = neg.astype(jnp.int32).reshape(B * K)
    in_pk = _tc_pack(in_embed)
    out_pk = _tc_pack(out_embed)
    pacc, nacc = _make_sc_scores()(center, pos, neg_flat, in_pk, out_pk)
    return _tc_loss(pacc, nacc)[0, 0]
